# baseline (speedup 1.0000x reference)
import functools

import jax
import jax.numpy as jnp
from jax import lax
from jax.experimental import pallas as pl
from jax.experimental.pallas import tpu as pltpu
from jax.experimental.pallas import tpu_sc as plsc

_B = 4096
_E = 1000
_NW = 32
_RPW = _B // _NW


def _make_at_gather():
    mesh = plsc.VectorSubcoreMesh(core_axis_name="c", subcore_axis_name="s")

    @functools.partial(
        pl.kernel,
        mesh=mesh,
        out_type=jax.ShapeDtypeStruct((_B,), jnp.float32),
        scratch_types=[
            pltpu.VMEM((_RPW,), jnp.int32),
            pltpu.VMEM((_RPW,), jnp.int32),
            pltpu.VMEM((_RPW,), jnp.float32),
            pltpu.SemaphoreType.DMA,
        ],
    )
    def at_gather(outflat_hbm, tgt_hbm, at_hbm, tgt_v, idx_v, at_v, sem):
        wid = lax.axis_index("s") * 2 + lax.axis_index("c")
        base = wid * _RPW
        pltpu.sync_copy(tgt_hbm.at[pl.ds(base, _RPW)], tgt_v)
        for c in range(_RPW // 16):
            t = tgt_v[pl.ds(c * 16, 16)]
            cols = base + c * 16 + lax.iota(jnp.int32, 16)
            idx_v[pl.ds(c * 16, 16)] = t * _B + cols
        pltpu.async_copy(outflat_hbm.at[idx_v], at_v, sem).wait()
        pltpu.sync_copy(at_v, at_hbm.at[pl.ds(base, _RPW)])

    return at_gather


_AT_GATHER = _make_at_gather()


def kernel(output, target):
    flat = output.T.reshape(_B * _E)
    at = _AT_GATHER(flat, target.astype(jnp.int32))
    return at[0]


# speedup vs baseline: 4.4550x; 4.4550x over previous
import jax
import jax.numpy as jnp
from jax.experimental import pallas as pl
from jax.experimental.pallas import tpu as pltpu

_B = 4096
_E = 1000
_BL = 1024
_MARGIN = 0.9


def _loss_body(out_ref, tgt_ref, acc_ref, vacc_ref):
    i = pl.program_id(0)

    @pl.when(i == 0)
    def _():
        vacc_ref[...] = jnp.zeros((8, _BL), jnp.float32)

    out = out_ref[...]
    tgt = tgt_ref[...].reshape(1, _BL)
    cls = jax.lax.broadcasted_iota(jnp.int32, (_E, _BL), 0)
    at = jnp.sum(jnp.where(cls == tgt, out, 0.0), axis=0, keepdims=True)
    d = jnp.maximum((_MARGIN - at) + out, 0.0)
    vacc_ref[...] += jnp.sum((d * d).reshape(_E // 8, 8, _BL), axis=0)

    @pl.when(i == pl.num_programs(0) - 1)
    def _():
        acc_ref[...] = jnp.full((1, 1), jnp.sum(vacc_ref[...]) * (1.0 / _B),
                                jnp.float32)


def kernel(output, target):
    out_t = output.T
    acc = pl.pallas_call(
        _loss_body,
        grid=(_B // _BL,),
        in_specs=[
            pl.BlockSpec((_E, _BL), lambda i: (0, i)),
            pl.BlockSpec((_BL,), lambda i: (i,)),
        ],
        out_specs=pl.BlockSpec((1, 1), lambda i: (0, 0)),
        out_shape=jax.ShapeDtypeStruct((1, 1), jnp.float32),
        scratch_shapes=[pltpu.VMEM((8, _BL), jnp.float32)],
    )(out_t, target.astype(jnp.int32))
    return acc[0, 0]
